# baseline (speedup 1.0000x reference)
import functools

import jax
import jax.numpy as jnp
from jax import lax
from jax.experimental import pallas as pl
from jax.experimental.pallas import tpu as pltpu
from jax.experimental.pallas import tpu_sc as plsc


def kernel(diffusion_step, pe_mat):
    (B,) = diffusion_step.shape
    V, D = pe_mat.shape

    info = plsc.get_sparse_core_info()
    NC, NS = info.num_cores, info.num_subcores
    NW = NC * NS
    b_per_w = B // NW
    STAGE_TILES = 8
    v_per_stage = (V // STAGE_TILES) // 8 * 8

    mesh = plsc.VectorSubcoreMesh(core_axis_name="c", subcore_axis_name="s")

    @functools.partial(
        pl.kernel,
        mesh=mesh,
        out_type=jax.ShapeDtypeStruct((B, D), jnp.float32),
        scratch_types=[
            pltpu.VMEM((b_per_w,), jnp.int32),
            pltpu.VMEM((b_per_w, D), jnp.float32),
            pltpu.VMEM_SHARED((V, D), jnp.float32),
            pltpu.SemaphoreType.DMA,
        ],
    )
    def gather_rows(idx_hbm, table_hbm, out_hbm, idx_v, rows_v, table_sh, sem):
        sid = lax.axis_index("s")
        wid = sid * NC + lax.axis_index("c")
        base = wid * b_per_w

        @pl.when(sid < STAGE_TILES)
        def _stage_table():
            row0 = sid * v_per_stage
            pltpu.sync_copy(
                table_hbm.at[pl.ds(row0, v_per_stage)],
                table_sh.at[pl.ds(row0, v_per_stage)],
            )

        rem = V - STAGE_TILES * v_per_stage
        if rem:

            @pl.when(sid == STAGE_TILES)
            def _stage_rem():
                pltpu.sync_copy(
                    table_hbm.at[pl.ds(STAGE_TILES * v_per_stage, rem)],
                    table_sh.at[pl.ds(STAGE_TILES * v_per_stage, rem)],
                )

        pltpu.sync_copy(idx_hbm.at[pl.ds(base, b_per_w)], idx_v)
        plsc.subcore_barrier()
        pltpu.async_copy(table_sh.at[idx_v], rows_v, sem).wait()
        pltpu.sync_copy(rows_v, out_hbm.at[pl.ds(base, b_per_w)])

    return gather_rows(diffusion_step, pe_mat)


# speedup vs baseline: 1.0148x; 1.0148x over previous
import functools

import jax
import jax.numpy as jnp
from jax import lax
from jax.experimental import pallas as pl
from jax.experimental.pallas import tpu as pltpu
from jax.experimental.pallas import tpu_sc as plsc


def kernel(diffusion_step, pe_mat):
    (B,) = diffusion_step.shape
    V, D = pe_mat.shape

    info = plsc.get_sparse_core_info()
    NC, NS = info.num_cores, info.num_subcores
    NW = NC * NS
    b_per_w = B // NW
    CHUNK = 64
    n_chunks = b_per_w // CHUNK
    STAGE_TILES = 8
    v_per_stage = (V // STAGE_TILES) // 8 * 8

    mesh = plsc.VectorSubcoreMesh(core_axis_name="c", subcore_axis_name="s")

    @functools.partial(
        pl.kernel,
        mesh=mesh,
        out_type=jax.ShapeDtypeStruct((B, D), jnp.float32),
        scratch_types=[
            pltpu.VMEM((b_per_w,), jnp.int32),
            pltpu.VMEM((b_per_w, D), jnp.float32),
            pltpu.VMEM_SHARED((V, D), jnp.float32),
        ]
        + [pltpu.SemaphoreType.DMA] * (n_chunks + 1),
    )
    def gather_rows(idx_hbm, table_hbm, out_hbm, idx_v, rows_v, table_sh, *sems):
        gsems, osem = sems[:n_chunks], sems[n_chunks]
        sid = lax.axis_index("s")
        wid = sid * NC + lax.axis_index("c")
        base = wid * b_per_w

        @pl.when(sid < STAGE_TILES)
        def _stage_table():
            row0 = sid * v_per_stage
            pltpu.sync_copy(
                table_hbm.at[pl.ds(row0, v_per_stage)],
                table_sh.at[pl.ds(row0, v_per_stage)],
            )

        rem = V - STAGE_TILES * v_per_stage
        if rem:

            @pl.when(sid == STAGE_TILES)
            def _stage_rem():
                pltpu.sync_copy(
                    table_hbm.at[pl.ds(STAGE_TILES * v_per_stage, rem)],
                    table_sh.at[pl.ds(STAGE_TILES * v_per_stage, rem)],
                )

        pltpu.sync_copy(idx_hbm.at[pl.ds(base, b_per_w)], idx_v)
        HEAD = 2
        gathers = [
            pltpu.async_copy(
                table_hbm.at[idx_v.at[pl.ds(j * CHUNK, CHUNK)]],
                rows_v.at[pl.ds(j * CHUNK, CHUNK)],
                gsems[j],
            )
            for j in range(HEAD)
        ]
        plsc.subcore_barrier()
        gathers += [
            pltpu.async_copy(
                table_sh.at[idx_v.at[pl.ds(j * CHUNK, CHUNK)]],
                rows_v.at[pl.ds(j * CHUNK, CHUNK)],
                gsems[j],
            )
            for j in range(HEAD, n_chunks)
        ]
        outs = []
        for j in range(n_chunks):
            gathers[j].wait()
            outs.append(
                pltpu.async_copy(
                    rows_v.at[pl.ds(j * CHUNK, CHUNK)],
                    out_hbm.at[pl.ds(base + j * CHUNK, CHUNK)],
                    osem,
                )
            )
        for o in outs:
            o.wait()

    return gather_rows(diffusion_step, pe_mat)


# speedup vs baseline: 1.0534x; 1.0381x over previous
import functools

import jax
import jax.numpy as jnp
from jax import lax
from jax.experimental import pallas as pl
from jax.experimental.pallas import tpu as pltpu
from jax.experimental.pallas import tpu_sc as plsc


def kernel(diffusion_step, pe_mat):
    (B,) = diffusion_step.shape
    V, D = pe_mat.shape

    info = plsc.get_sparse_core_info()
    NC, NS = info.num_cores, info.num_subcores
    NW = NC * NS
    b_per_w = B // NW
    CHUNK = 64
    n_chunks = b_per_w // CHUNK
    STAGE_TILES = 8
    v_per_stage = (V // STAGE_TILES) // 8 * 8

    mesh = plsc.VectorSubcoreMesh(core_axis_name="c", subcore_axis_name="s")

    @functools.partial(
        pl.kernel,
        mesh=mesh,
        out_type=jax.ShapeDtypeStruct((B, D), jnp.float32),
        scratch_types=[
            pltpu.VMEM((b_per_w,), jnp.int32),
            pltpu.VMEM((b_per_w, D), jnp.float32),
            pltpu.VMEM_SHARED((V, D), jnp.float32),
        ]
        + [pltpu.SemaphoreType.DMA] * (n_chunks + 1),
    )
    def gather_rows(idx_hbm, table_hbm, out_hbm, idx_v, rows_v, table_sh, *sems):
        gsems, osem = sems[:n_chunks], sems[n_chunks]
        sid = lax.axis_index("s")
        wid = sid * NC + lax.axis_index("c")
        base = wid * b_per_w

        @pl.when(sid < STAGE_TILES)
        def _stage_table():
            row0 = sid * v_per_stage
            pltpu.sync_copy(
                table_hbm.at[pl.ds(row0, v_per_stage)],
                table_sh.at[pl.ds(row0, v_per_stage)],
            )

        rem = V - STAGE_TILES * v_per_stage
        if rem:

            @pl.when(sid == STAGE_TILES)
            def _stage_rem():
                pltpu.sync_copy(
                    table_hbm.at[pl.ds(STAGE_TILES * v_per_stage, rem)],
                    table_sh.at[pl.ds(STAGE_TILES * v_per_stage, rem)],
                )

        pltpu.sync_copy(idx_hbm.at[pl.ds(base, b_per_w)], idx_v)
        plsc.subcore_barrier()
        gathers = [
            pltpu.async_copy(
                table_sh.at[idx_v.at[pl.ds(j * CHUNK, CHUNK)]],
                rows_v.at[pl.ds(j * CHUNK, CHUNK)],
                gsems[j],
            )
            for j in range(n_chunks)
        ]
        outs = []
        for j in range(n_chunks):
            gathers[j].wait()
            outs.append(
                pltpu.async_copy(
                    rows_v.at[pl.ds(j * CHUNK, CHUNK)],
                    out_hbm.at[pl.ds(base + j * CHUNK, CHUNK)],
                    osem,
                )
            )
        for o in outs:
            o.wait()

    return gather_rows(diffusion_step, pe_mat)
